# Initial kernel scaffold; baseline (speedup 1.0000x reference)
import jax
import jax.numpy as jnp
from jax.experimental import pallas as pl


def kernel(vfeat, efeat, degE, degV, vfeat0, W, node_idx, hedge_idx, alpha, beta):
    raise NotImplementedError("write your pallas kernel here")


# speedup vs baseline: 21.5791x; 21.5791x over previous
import functools

import jax
import jax.numpy as jnp
from jax import lax
from jax.experimental import pallas as pl
from jax.experimental.pallas import tpu as pltpu
from jax.experimental.pallas import tpu_sc as plsc

N = 10000
M = 5000
E = 320000
D = 128
MP = 5120
NP = 10240
C = 128
NCHUNK = E // C
NW = 32
BASE = NCHUNK // NW
EXTRA = NCHUNK - BASE * NW
NPAIR = BASE // 2

_mesh = plsc.VectorSubcoreMesh(core_axis_name="c", subcore_axis_name="s")


@functools.partial(
    pl.kernel,
    mesh=_mesh,
    out_type=[
        jax.ShapeDtypeStruct((2, MP, D), jnp.float32),
        jax.ShapeDtypeStruct((2 * MP,), jnp.float32),
    ],
    scratch_types=[
        pltpu.VMEM((2, 2, C), jnp.int32),
        pltpu.VMEM((2, C, D), jnp.float32),
        pltpu.VMEM((C,), jnp.float32),
        pltpu.VMEM((MP // 16,), jnp.float32),
        pltpu.VMEM_SHARED((MP, D), jnp.float32),
        pltpu.VMEM_SHARED((MP,), jnp.float32),
        pltpu.SemaphoreType.DMA, pltpu.SemaphoreType.DMA,
        pltpu.SemaphoreType.DMA, pltpu.SemaphoreType.DMA,
        pltpu.SemaphoreType.DMA, pltpu.SemaphoreType.DMA,
        pltpu.SemaphoreType.DMA, pltpu.SemaphoreType.DMA,
    ],
)
def _sc_phase1(vfeat, idx2, z2d, z1d, ones, sums_out, cnt_out,
               idx_v, rows_v, ones_v, cnt_b, acc_sp, cnt_sp,
               si0, si1, sg0, sg1, ss0, ss1, sc0, sc1):
    cid = lax.axis_index("c")
    sid = lax.axis_index("s")
    wid = sid * 2 + cid
    rpt = MP // 16
    r0 = sid * rpt
    si = (si0, si1)
    sg = (sg0, sg1)
    ss = (ss0, ss1)
    sc = (sc0, sc1)

    pltpu.sync_copy(z2d.at[pl.ds(0, rpt)], acc_sp.at[pl.ds(r0, rpt)])
    pltpu.sync_copy(z1d.at[pl.ds(r0, rpt)], cnt_b)
    pltpu.sync_copy(cnt_b, cnt_sp.at[pl.ds(r0, rpt)])
    pltpu.sync_copy(ones, ones_v)
    plsc.subcore_barrier()

    base0 = wid * BASE

    def wait_scatters(j):
        pltpu.make_async_copy(rows_v.at[j], acc_sp.at[idx_v.at[j, 1]], ss[j]).wait()
        pltpu.make_async_copy(ones_v, cnt_sp.at[idx_v.at[j, 1]], sc[j]).wait()

    def body(k2, carry):
        c0 = base0 + 2 * k2

        @pl.when(k2 > 0)
        def _():
            wait_scatters(0)
            wait_scatters(1)

        h0 = pltpu.async_copy(idx2.at[c0], idx_v.at[0], si[0])
        h1 = pltpu.async_copy(idx2.at[c0 + 1], idx_v.at[1], si[1])
        h0.wait()
        g0 = pltpu.async_copy(vfeat.at[idx_v.at[0, 0]], rows_v.at[0], sg[0])
        h1.wait()
        g1 = pltpu.async_copy(vfeat.at[idx_v.at[1, 0]], rows_v.at[1], sg[1])
        g0.wait()
        pltpu.async_copy(rows_v.at[0], acc_sp.at[idx_v.at[0, 1]], ss[0], add=True)
        pltpu.async_copy(ones_v, cnt_sp.at[idx_v.at[0, 1]], sc[0], add=True)
        g1.wait()
        pltpu.async_copy(rows_v.at[1], acc_sp.at[idx_v.at[1, 1]], ss[1], add=True)
        pltpu.async_copy(ones_v, cnt_sp.at[idx_v.at[1, 1]], sc[1], add=True)
        return carry

    lax.fori_loop(0, NPAIR, body, 0)
    wait_scatters(0)
    wait_scatters(1)

    @pl.when(wid < EXTRA)
    def _():
        c = NW * BASE + wid
        pltpu.sync_copy(idx2.at[c], idx_v.at[0])
        pltpu.async_copy(vfeat.at[idx_v.at[0, 0]], rows_v.at[0], sg[0]).wait()
        pltpu.sync_copy(rows_v.at[0], acc_sp.at[idx_v.at[0, 1]], add=True)
        pltpu.sync_copy(ones_v, cnt_sp.at[idx_v.at[0, 1]], add=True)

    plsc.subcore_barrier()
    pltpu.sync_copy(acc_sp.at[pl.ds(r0, rpt)], sums_out.at[cid, pl.ds(r0, rpt)])
    pltpu.sync_copy(cnt_sp.at[pl.ds(r0, rpt)], cnt_b)
    pltpu.sync_copy(cnt_b, cnt_out.at[pl.ds(cid * MP + r0, rpt)])


@functools.partial(
    pl.kernel,
    mesh=_mesh,
    out_type=jax.ShapeDtypeStruct((2, NP, D), jnp.float32),
    scratch_types=[
        pltpu.VMEM((2, 2, C), jnp.int32),
        pltpu.VMEM((2, C, D), jnp.float32),
        pltpu.VMEM_SHARED((NP, D), jnp.float32),
        pltpu.SemaphoreType.DMA, pltpu.SemaphoreType.DMA,
        pltpu.SemaphoreType.DMA, pltpu.SemaphoreType.DMA,
        pltpu.SemaphoreType.DMA, pltpu.SemaphoreType.DMA,
    ],
)
def _sc_phase2(etmp, idx2, z2d, vagg_out, idx_v, rows_v, acc_sp,
               si0, si1, sg0, sg1, ss0, ss1):
    cid = lax.axis_index("c")
    sid = lax.axis_index("s")
    wid = sid * 2 + cid
    rpt = NP // 16
    r0 = sid * rpt
    si = (si0, si1)
    sg = (sg0, sg1)
    ss = (ss0, ss1)

    pltpu.sync_copy(z2d.at[pl.ds(0, rpt)], acc_sp.at[pl.ds(r0, rpt)])
    plsc.subcore_barrier()

    base0 = wid * BASE

    def wait_scatter(j):
        pltpu.make_async_copy(rows_v.at[j], acc_sp.at[idx_v.at[j, 0]], ss[j]).wait()

    def body(k2, carry):
        c0 = base0 + 2 * k2

        @pl.when(k2 > 0)
        def _():
            wait_scatter(0)
            wait_scatter(1)

        h0 = pltpu.async_copy(idx2.at[c0], idx_v.at[0], si[0])
        h1 = pltpu.async_copy(idx2.at[c0 + 1], idx_v.at[1], si[1])
        h0.wait()
        g0 = pltpu.async_copy(etmp.at[idx_v.at[0, 1]], rows_v.at[0], sg[0])
        h1.wait()
        g1 = pltpu.async_copy(etmp.at[idx_v.at[1, 1]], rows_v.at[1], sg[1])
        g0.wait()
        pltpu.async_copy(rows_v.at[0], acc_sp.at[idx_v.at[0, 0]], ss[0], add=True)
        g1.wait()
        pltpu.async_copy(rows_v.at[1], acc_sp.at[idx_v.at[1, 0]], ss[1], add=True)
        return carry

    lax.fori_loop(0, NPAIR, body, 0)
    wait_scatter(0)
    wait_scatter(1)

    @pl.when(wid < EXTRA)
    def _():
        c = NW * BASE + wid
        pltpu.sync_copy(idx2.at[c], idx_v.at[0])
        pltpu.async_copy(etmp.at[idx_v.at[0, 1]], rows_v.at[0], sg[0]).wait()
        pltpu.sync_copy(rows_v.at[0], acc_sp.at[idx_v.at[0, 0]], add=True)

    plsc.subcore_barrier()
    pltpu.sync_copy(acc_sp.at[pl.ds(r0, rpt)], vagg_out.at[cid, pl.ds(r0, rpt)])


def _tc1_body(sums_ref, cnt_ref, dege_ref, ef_ref, etmp_ref):
    s = sums_ref[0] + sums_ref[1]
    c = cnt_ref[0] + cnt_ref[1]
    ef = s / jnp.maximum(c, 1.0)
    ef_ref[...] = ef
    etmp_ref[...] = ef * dege_ref[...]


_tc1 = pl.pallas_call(
    _tc1_body,
    out_shape=[
        jax.ShapeDtypeStruct((MP, D), jnp.float32),
        jax.ShapeDtypeStruct((MP, D), jnp.float32),
    ],
)

RB = 1000


def _tc2_body(vagg_ref, degv_ref, vf0_ref, w_ref, ab_ref, out_ref):
    a = ab_ref[0:1, 0:1]
    b = ab_ref[0:1, 1:2]
    va = vagg_ref[0] + vagg_ref[1]
    vi = (1.0 - a) * degv_ref[...] * va + a * vf0_ref[...]
    vw = lax.dot_general(vi, w_ref[...], (((1,), (1,)), ((), ())),
                         preferred_element_type=jnp.float32,
                         precision=lax.Precision.HIGHEST)
    out_ref[...] = (1.0 - b) * vi + b * vw


_tc2 = pl.pallas_call(
    _tc2_body,
    grid=(N // RB,),
    in_specs=[
        pl.BlockSpec((2, RB, D), lambda i: (0, i, 0)),
        pl.BlockSpec((RB, 1), lambda i: (i, 0)),
        pl.BlockSpec((RB, D), lambda i: (i, 0)),
        pl.BlockSpec((D, D), lambda i: (0, 0)),
        pl.BlockSpec((1, 2), lambda i: (0, 0)),
    ],
    out_specs=pl.BlockSpec((RB, D), lambda i: (i, 0)),
    out_shape=jax.ShapeDtypeStruct((N, D), jnp.float32),
)


def kernel(vfeat, efeat, degE, degV, vfeat0, W, node_idx, hedge_idx, alpha, beta):
    del efeat
    idx2 = jnp.stack(
        [node_idx.reshape(NCHUNK, C), hedge_idx.reshape(NCHUNK, C)], axis=1)
    z2d = jnp.zeros((NP // 16, D), jnp.float32)
    z1d = jnp.zeros((MP,), jnp.float32)
    ones = jnp.ones((C,), jnp.float32)

    sums, cnt = _sc_phase1(vfeat, idx2, z2d, z1d, ones)

    dege_col = jnp.concatenate(
        [degE, jnp.zeros((MP - M,), jnp.float32)]).reshape(MP, 1)
    ef_pad, etmp_pad = _tc1(sums, cnt.reshape(2, MP, 1), dege_col)

    vagg = _sc_phase2(etmp_pad, idx2, z2d)

    ab = jnp.stack([alpha, beta]).astype(jnp.float32).reshape(1, 2)
    v = _tc2(vagg, degV.reshape(N, 1), vfeat0, W, ab)
    return (v, ef_pad[:M])


# speedup vs baseline: 26.0062x; 1.2052x over previous
import functools

import jax
import jax.numpy as jnp
from jax import lax
from jax.experimental import pallas as pl
from jax.experimental.pallas import tpu as pltpu
from jax.experimental.pallas import tpu_sc as plsc

N = 10000
M = 5000
E = 320000
D = 128
MP = 5120
NP = 10240
C = 128
NCHUNK = E // C
NW = 32
BASE = NCHUNK // NW
EXTRA = NCHUNK - BASE * NW
NBUF = 3
NBLK = BASE // NBUF
NPAIR = BASE // 2

_mesh = plsc.VectorSubcoreMesh(core_axis_name="c", subcore_axis_name="s")


@functools.partial(
    pl.kernel,
    mesh=_mesh,
    out_type=[
        jax.ShapeDtypeStruct((2, MP, D), jnp.float32),
        jax.ShapeDtypeStruct((2 * MP,), jnp.float32),
    ],
    scratch_types=[
        pltpu.VMEM((BASE + 1, 2, C), jnp.int32),
        pltpu.VMEM((NBUF, C, D), jnp.float32),
        pltpu.VMEM((C,), jnp.float32),
        pltpu.VMEM((MP // 16,), jnp.float32),
        pltpu.VMEM_SHARED((MP, D), jnp.float32),
        pltpu.VMEM_SHARED((MP,), jnp.float32),
    ] + [pltpu.SemaphoreType.DMA] * (3 * NBUF),
)
def _sc_phase1(vfeat, idx2, z2d, z1d, ones, sums_out, cnt_out,
               idx_a, rows_v, ones_v, cnt_b, acc_sp, cnt_sp, *sems):
    cid = lax.axis_index("c")
    sid = lax.axis_index("s")
    wid = sid * 2 + cid
    rpt = MP // 16
    r0 = sid * rpt
    sg = sems[:NBUF]
    ss = sems[NBUF:2 * NBUF]
    sc = sems[2 * NBUF:]

    base0 = wid * BASE
    pltpu.sync_copy(idx2.at[pl.ds(base0, BASE)], idx_a.at[pl.ds(0, BASE)])
    pltpu.sync_copy(z2d.at[pl.ds(0, rpt)], acc_sp.at[pl.ds(r0, rpt)])
    pltpu.sync_copy(z1d.at[pl.ds(r0, rpt)], cnt_b)
    pltpu.sync_copy(cnt_b, cnt_sp.at[pl.ds(r0, rpt)])
    pltpu.sync_copy(ones, ones_v)
    plsc.subcore_barrier()

    def start_gather(k, j):
        pltpu.async_copy(vfeat.at[idx_a.at[k, 0]], rows_v.at[j], sg[j])

    def wait_gather(j):
        pltpu.make_async_copy(vfeat.at[idx_a.at[0, 0]], rows_v.at[j], sg[j]).wait()

    def wait_scatters(j):
        pltpu.make_async_copy(rows_v.at[j], acc_sp.at[idx_a.at[0, 1]], ss[j]).wait()
        pltpu.make_async_copy(ones_v, cnt_sp.at[idx_a.at[0, 1]], sc[j]).wait()

    for k in range(NBUF - 1):
        start_gather(k, k)

    def body(b, carry):
        i0 = b * NBUF
        for j in range(NBUF):
            i = i0 + j
            wait_gather(j)
            pltpu.async_copy(rows_v.at[j], acc_sp.at[idx_a.at[i, 1]], ss[j], add=True)
            pltpu.async_copy(ones_v, cnt_sp.at[idx_a.at[i, 1]], sc[j], add=True)
            jn = (j + NBUF - 1) % NBUF
            if j == 0:
                @pl.when(b > 0)
                def _():
                    wait_scatters(jn)
            else:
                wait_scatters(jn)

            @pl.when(i + NBUF - 1 < BASE)
            def _():
                start_gather(i + NBUF - 1, jn)

        return carry

    lax.fori_loop(0, NBLK, body, 0)
    wait_scatters((BASE - 1) % NBUF)

    @pl.when(wid < EXTRA)
    def _():
        pltpu.sync_copy(idx2.at[NW * BASE + wid], idx_a.at[BASE])
        pltpu.async_copy(vfeat.at[idx_a.at[BASE, 0]], rows_v.at[0], sg[0]).wait()
        pltpu.sync_copy(rows_v.at[0], acc_sp.at[idx_a.at[BASE, 1]], add=True)
        pltpu.sync_copy(ones_v, cnt_sp.at[idx_a.at[BASE, 1]], add=True)

    plsc.subcore_barrier()
    pltpu.sync_copy(acc_sp.at[pl.ds(r0, rpt)], sums_out.at[cid, pl.ds(r0, rpt)])
    pltpu.sync_copy(cnt_sp.at[pl.ds(r0, rpt)], cnt_b)
    pltpu.sync_copy(cnt_b, cnt_out.at[pl.ds(cid * MP + r0, rpt)])


@functools.partial(
    pl.kernel,
    mesh=_mesh,
    out_type=jax.ShapeDtypeStruct((2, NP, D), jnp.float32),
    scratch_types=[
        pltpu.VMEM((2, 2, C), jnp.int32),
        pltpu.VMEM((2, C, D), jnp.float32),
        pltpu.VMEM_SHARED((NP, D), jnp.float32),
        pltpu.SemaphoreType.DMA, pltpu.SemaphoreType.DMA,
        pltpu.SemaphoreType.DMA, pltpu.SemaphoreType.DMA,
        pltpu.SemaphoreType.DMA, pltpu.SemaphoreType.DMA,
    ],
)
def _sc_phase2(etmp, idx2, z2d, vagg_out, idx_v, rows_v, acc_sp,
               si0, si1, sg0, sg1, ss0, ss1):
    cid = lax.axis_index("c")
    sid = lax.axis_index("s")
    wid = sid * 2 + cid
    rpt = NP // 16
    r0 = sid * rpt
    si = (si0, si1)
    sg = (sg0, sg1)
    ss = (ss0, ss1)

    pltpu.sync_copy(z2d.at[pl.ds(0, rpt)], acc_sp.at[pl.ds(r0, rpt)])
    plsc.subcore_barrier()

    base0 = wid * BASE

    def wait_scatter(j):
        pltpu.make_async_copy(rows_v.at[j], acc_sp.at[idx_v.at[j, 0]], ss[j]).wait()

    def body(k2, carry):
        c0 = base0 + 2 * k2

        @pl.when(k2 > 0)
        def _():
            wait_scatter(0)
            wait_scatter(1)

        h0 = pltpu.async_copy(idx2.at[c0], idx_v.at[0], si[0])
        h1 = pltpu.async_copy(idx2.at[c0 + 1], idx_v.at[1], si[1])
        h0.wait()
        g0 = pltpu.async_copy(etmp.at[idx_v.at[0, 1]], rows_v.at[0], sg[0])
        h1.wait()
        g1 = pltpu.async_copy(etmp.at[idx_v.at[1, 1]], rows_v.at[1], sg[1])
        g0.wait()
        pltpu.async_copy(rows_v.at[0], acc_sp.at[idx_v.at[0, 0]], ss[0], add=True)
        g1.wait()
        pltpu.async_copy(rows_v.at[1], acc_sp.at[idx_v.at[1, 0]], ss[1], add=True)
        return carry

    lax.fori_loop(0, NPAIR, body, 0)
    wait_scatter(0)
    wait_scatter(1)

    @pl.when(wid < EXTRA)
    def _():
        c = NW * BASE + wid
        pltpu.sync_copy(idx2.at[c], idx_v.at[0])
        pltpu.async_copy(etmp.at[idx_v.at[0, 1]], rows_v.at[0], sg[0]).wait()
        pltpu.sync_copy(rows_v.at[0], acc_sp.at[idx_v.at[0, 0]], add=True)

    plsc.subcore_barrier()
    pltpu.sync_copy(acc_sp.at[pl.ds(r0, rpt)], vagg_out.at[cid, pl.ds(r0, rpt)])


def _tc1_body(sums_ref, cnt_ref, dege_ref, ef_ref, etmp_ref):
    s = sums_ref[0] + sums_ref[1]
    c = cnt_ref[0] + cnt_ref[1]
    ef = s / jnp.maximum(c, 1.0)
    ef_ref[...] = ef
    etmp_ref[...] = ef * dege_ref[...]


_tc1 = pl.pallas_call(
    _tc1_body,
    out_shape=[
        jax.ShapeDtypeStruct((MP, D), jnp.float32),
        jax.ShapeDtypeStruct((MP, D), jnp.float32),
    ],
)

RB = 1000


def _tc2_body(vagg_ref, degv_ref, vf0_ref, w_ref, ab_ref, out_ref):
    a = ab_ref[0:1, 0:1]
    b = ab_ref[0:1, 1:2]
    va = vagg_ref[0] + vagg_ref[1]
    vi = (1.0 - a) * degv_ref[...] * va + a * vf0_ref[...]
    vw = lax.dot_general(vi, w_ref[...], (((1,), (1,)), ((), ())),
                         preferred_element_type=jnp.float32,
                         precision=lax.Precision.HIGHEST)
    out_ref[...] = (1.0 - b) * vi + b * vw


_tc2 = pl.pallas_call(
    _tc2_body,
    grid=(N // RB,),
    in_specs=[
        pl.BlockSpec((2, RB, D), lambda i: (0, i, 0)),
        pl.BlockSpec((RB, 1), lambda i: (i, 0)),
        pl.BlockSpec((RB, D), lambda i: (i, 0)),
        pl.BlockSpec((D, D), lambda i: (0, 0)),
        pl.BlockSpec((1, 2), lambda i: (0, 0)),
    ],
    out_specs=pl.BlockSpec((RB, D), lambda i: (i, 0)),
    out_shape=jax.ShapeDtypeStruct((N, D), jnp.float32),
)


def kernel(vfeat, efeat, degE, degV, vfeat0, W, node_idx, hedge_idx, alpha, beta):
    del efeat
    idx2 = jnp.stack(
        [node_idx.reshape(NCHUNK, C), hedge_idx.reshape(NCHUNK, C)], axis=1)
    z2d = jnp.zeros((NP // 16, D), jnp.float32)
    z1d = jnp.zeros((MP,), jnp.float32)
    ones = jnp.ones((C,), jnp.float32)

    sums, cnt = _sc_phase1(vfeat, idx2, z2d, z1d, ones)

    dege_col = jnp.concatenate(
        [degE, jnp.zeros((MP - M,), jnp.float32)]).reshape(MP, 1)
    ef_pad, etmp_pad = _tc1(sums, cnt.reshape(2, MP, 1), dege_col)

    vagg = _sc_phase2(etmp_pad, idx2, z2d)

    ab = jnp.stack([alpha, beta]).astype(jnp.float32).reshape(1, 2)
    v = _tc2(vagg, degV.reshape(N, 1), vfeat0, W, ab)
    return (v, ef_pad[:M])


# speedup vs baseline: 29.0759x; 1.1180x over previous
import functools

import jax
import jax.numpy as jnp
from jax import lax
from jax.experimental import pallas as pl
from jax.experimental.pallas import tpu as pltpu
from jax.experimental.pallas import tpu_sc as plsc

N = 10000
M = 5000
E = 320000
D = 128
MP = 5120
NP = 10240
C = 128
NCHUNK = E // C
NW = 32
BASE = NCHUNK // NW
EXTRA = NCHUNK - BASE * NW
NBUF = 3
NBLK = BASE // NBUF
NPAIR = BASE // 2

_mesh = plsc.VectorSubcoreMesh(core_axis_name="c", subcore_axis_name="s")


@functools.partial(
    pl.kernel,
    mesh=_mesh,
    out_type=[
        jax.ShapeDtypeStruct((2, MP, D), jnp.float32),
        jax.ShapeDtypeStruct((2 * MP,), jnp.float32),
    ],
    scratch_types=[
        pltpu.VMEM((BASE + 1, 2, C), jnp.int32),
        pltpu.VMEM((NBUF, C, D), jnp.float32),
        pltpu.VMEM((C,), jnp.float32),
        pltpu.VMEM((MP // 16,), jnp.float32),
        pltpu.VMEM_SHARED((MP, D), jnp.float32),
        pltpu.VMEM_SHARED((MP,), jnp.float32),
    ] + [pltpu.SemaphoreType.DMA] * (3 * NBUF),
)
def _sc_phase1(vfeat, idx2, z2d, z1d, ones, sums_out, cnt_out,
               idx_a, rows_v, ones_v, cnt_b, acc_sp, cnt_sp, *sems):
    cid = lax.axis_index("c")
    sid = lax.axis_index("s")
    wid = sid * 2 + cid
    rpt = MP // 16
    r0 = sid * rpt
    sg = sems[:NBUF]
    ss = sems[NBUF:2 * NBUF]
    sc = sems[2 * NBUF:]

    base0 = wid * BASE
    pltpu.sync_copy(idx2.at[pl.ds(base0, BASE)], idx_a.at[pl.ds(0, BASE)])
    pltpu.sync_copy(z2d.at[pl.ds(0, rpt)], acc_sp.at[pl.ds(r0, rpt)])
    pltpu.sync_copy(z1d.at[pl.ds(r0, rpt)], cnt_b)
    pltpu.sync_copy(cnt_b, cnt_sp.at[pl.ds(r0, rpt)])
    pltpu.sync_copy(ones, ones_v)
    plsc.subcore_barrier()

    def start_gather(k, j):
        pltpu.async_copy(vfeat.at[idx_a.at[k, 0]], rows_v.at[j], sg[j])

    def wait_gather(j):
        pltpu.make_async_copy(vfeat.at[idx_a.at[0, 0]], rows_v.at[j], sg[j]).wait()

    def wait_scatters(j):
        pltpu.make_async_copy(rows_v.at[j], acc_sp.at[idx_a.at[0, 1]], ss[j]).wait()
        pltpu.make_async_copy(ones_v, cnt_sp.at[idx_a.at[0, 1]], sc[j]).wait()

    for k in range(NBUF - 1):
        start_gather(k, k)

    def body(b, carry):
        i0 = b * NBUF
        for j in range(NBUF):
            i = i0 + j
            wait_gather(j)
            pltpu.async_copy(rows_v.at[j], acc_sp.at[idx_a.at[i, 1]], ss[j], add=True)
            pltpu.async_copy(ones_v, cnt_sp.at[idx_a.at[i, 1]], sc[j], add=True)
            jn = (j + NBUF - 1) % NBUF
            if j == 0:
                @pl.when(b > 0)
                def _():
                    wait_scatters(jn)
            else:
                wait_scatters(jn)

            @pl.when(i + NBUF - 1 < BASE)
            def _():
                start_gather(i + NBUF - 1, jn)

        return carry

    lax.fori_loop(0, NBLK, body, 0)
    wait_scatters((BASE - 1) % NBUF)

    @pl.when(wid < EXTRA)
    def _():
        pltpu.sync_copy(idx2.at[NW * BASE + wid], idx_a.at[BASE])
        pltpu.async_copy(vfeat.at[idx_a.at[BASE, 0]], rows_v.at[0], sg[0]).wait()
        pltpu.sync_copy(rows_v.at[0], acc_sp.at[idx_a.at[BASE, 1]], add=True)
        pltpu.sync_copy(ones_v, cnt_sp.at[idx_a.at[BASE, 1]], add=True)

    plsc.subcore_barrier()
    pltpu.sync_copy(acc_sp.at[pl.ds(r0, rpt)], sums_out.at[cid, pl.ds(r0, rpt)])
    pltpu.sync_copy(cnt_sp.at[pl.ds(r0, rpt)], cnt_b)
    pltpu.sync_copy(cnt_b, cnt_out.at[pl.ds(cid * MP + r0, rpt)])


@functools.partial(
    pl.kernel,
    mesh=_mesh,
    out_type=jax.ShapeDtypeStruct((2, NP, D), jnp.float32),
    scratch_types=[
        pltpu.VMEM((BASE + 1, 1, C), jnp.int32),
        pltpu.VMEM((2, 1, C), jnp.int32),
        pltpu.VMEM((2, C, D), jnp.float32),
        pltpu.VMEM_SHARED((NP, D), jnp.float32),
        pltpu.SemaphoreType.DMA, pltpu.SemaphoreType.DMA,
        pltpu.SemaphoreType.DMA, pltpu.SemaphoreType.DMA,
        pltpu.SemaphoreType.DMA, pltpu.SemaphoreType.DMA,
    ],
)
def _sc_phase2(etmp, idx_h, idx_n, z2d, vagg_out, hidx_a, sidx, rows_v, acc_sp,
               si0, si1, sg0, sg1, ss0, ss1):
    cid = lax.axis_index("c")
    sid = lax.axis_index("s")
    wid = sid * 2 + cid
    rpt = NP // 16
    r0 = sid * rpt
    si = (si0, si1)
    sg = (sg0, sg1)
    ss = (ss0, ss1)

    base0 = wid * BASE
    pltpu.sync_copy(idx_h.at[pl.ds(base0, BASE)], hidx_a.at[pl.ds(0, BASE)])
    pltpu.sync_copy(z2d.at[pl.ds(0, rpt)], acc_sp.at[pl.ds(r0, rpt)])
    plsc.subcore_barrier()

    def start_gather(k, j):
        pltpu.async_copy(etmp.at[hidx_a.at[k, 0]], rows_v.at[j], sg[j])

    def wait_gather(j):
        pltpu.make_async_copy(etmp.at[hidx_a.at[0, 0]], rows_v.at[j], sg[j]).wait()

    def wait_scatter(j):
        pltpu.make_async_copy(rows_v.at[j], acc_sp.at[sidx.at[j, 0]], ss[j]).wait()

    def wait_sidx(j):
        pltpu.make_async_copy(idx_n.at[0], sidx.at[j], si[j]).wait()

    pltpu.sync_copy(idx_n.at[base0], sidx.at[0])
    start_gather(0, 0)

    def body(b, carry):
        for j in range(2):
            i = 2 * b + j
            jn = 1 - j
            if j == 1:
                wait_sidx(1)
            else:
                @pl.when(b > 0)
                def _():
                    wait_sidx(0)
            wait_gather(j)
            pltpu.async_copy(rows_v.at[j], acc_sp.at[sidx.at[j, 0]], ss[j], add=True)
            if j == 0:
                @pl.when(b > 0)
                def _():
                    wait_scatter(1)

                pltpu.async_copy(idx_n.at[base0 + i + 1], sidx.at[1], si[1])
                start_gather(i + 1, 1)
            else:
                wait_scatter(0)

                @pl.when(b < NPAIR - 1)
                def _():
                    pltpu.async_copy(idx_n.at[base0 + i + 1], sidx.at[0], si[0])
                    start_gather(i + 1, 0)
        return carry

    lax.fori_loop(0, NPAIR, body, 0)
    wait_scatter((BASE - 1) % 2)

    @pl.when(wid < EXTRA)
    def _():
        c = NW * BASE + wid
        pltpu.sync_copy(idx_h.at[c], hidx_a.at[BASE])
        pltpu.sync_copy(idx_n.at[c], sidx.at[0])
        pltpu.async_copy(etmp.at[hidx_a.at[BASE, 0]], rows_v.at[0], sg[0]).wait()
        pltpu.sync_copy(rows_v.at[0], acc_sp.at[sidx.at[0, 0]], add=True)

    plsc.subcore_barrier()
    pltpu.sync_copy(acc_sp.at[pl.ds(r0, rpt)], vagg_out.at[cid, pl.ds(r0, rpt)])


def _tc1_body(sums_ref, cnt_ref, dege_ref, ef_ref, etmp_ref):
    s = sums_ref[0] + sums_ref[1]
    c = cnt_ref[0] + cnt_ref[1]
    ef = s / jnp.maximum(c, 1.0)
    ef_ref[...] = ef
    etmp_ref[...] = ef * dege_ref[...]


_tc1 = pl.pallas_call(
    _tc1_body,
    out_shape=[
        jax.ShapeDtypeStruct((MP, D), jnp.float32),
        jax.ShapeDtypeStruct((MP, D), jnp.float32),
    ],
)

RB = 1000


def _tc2_body(vagg_ref, degv_ref, vf0_ref, w_ref, ab_ref, out_ref):
    a = ab_ref[0:1, 0:1]
    b = ab_ref[0:1, 1:2]
    va = vagg_ref[0] + vagg_ref[1]
    vi = (1.0 - a) * degv_ref[...] * va + a * vf0_ref[...]
    vw = lax.dot_general(vi, w_ref[...], (((1,), (1,)), ((), ())),
                         preferred_element_type=jnp.float32,
                         precision=lax.Precision.HIGHEST)
    out_ref[...] = (1.0 - b) * vi + b * vw


_tc2 = pl.pallas_call(
    _tc2_body,
    grid=(N // RB,),
    in_specs=[
        pl.BlockSpec((2, RB, D), lambda i: (0, i, 0)),
        pl.BlockSpec((RB, 1), lambda i: (i, 0)),
        pl.BlockSpec((RB, D), lambda i: (i, 0)),
        pl.BlockSpec((D, D), lambda i: (0, 0)),
        pl.BlockSpec((1, 2), lambda i: (0, 0)),
    ],
    out_specs=pl.BlockSpec((RB, D), lambda i: (i, 0)),
    out_shape=jax.ShapeDtypeStruct((N, D), jnp.float32),
)


def kernel(vfeat, efeat, degE, degV, vfeat0, W, node_idx, hedge_idx, alpha, beta):
    del efeat
    idx2 = jnp.stack(
        [node_idx.reshape(NCHUNK, C), hedge_idx.reshape(NCHUNK, C)], axis=1)
    z2d = jnp.zeros((NP // 16, D), jnp.float32)
    z1d = jnp.zeros((MP,), jnp.float32)
    ones = jnp.ones((C,), jnp.float32)

    sums, cnt = _sc_phase1(vfeat, idx2, z2d, z1d, ones)

    dege_col = jnp.concatenate(
        [degE, jnp.zeros((MP - M,), jnp.float32)]).reshape(MP, 1)
    ef_pad, etmp_pad = _tc1(sums, cnt.reshape(2, MP, 1), dege_col)

    vagg = _sc_phase2(etmp_pad, hedge_idx.reshape(NCHUNK, 1, C),
                      node_idx.reshape(NCHUNK, 1, C), z2d)

    ab = jnp.stack([alpha, beta]).astype(jnp.float32).reshape(1, 2)
    v = _tc2(vagg, degV.reshape(N, 1), vfeat0, W, ab)
    return (v, ef_pad[:M])


# speedup vs baseline: 29.0901x; 1.0005x over previous
import functools

import jax
import jax.numpy as jnp
from jax import lax
from jax.experimental import pallas as pl
from jax.experimental.pallas import tpu as pltpu
from jax.experimental.pallas import tpu_sc as plsc

N = 10000
M = 5000
E = 320000
D = 128
MP = 5120
NP = 10240
C = 128
NCHUNK = E // C
NW = 32
BASE = NCHUNK // NW
EXTRA = NCHUNK - BASE * NW
NBUF = 3
NBLK = BASE // NBUF
NPAIR = BASE // 2

_mesh = plsc.VectorSubcoreMesh(core_axis_name="c", subcore_axis_name="s")


@functools.partial(
    pl.kernel,
    mesh=_mesh,
    out_type=[
        jax.ShapeDtypeStruct((2, MP, D), jnp.float32),
        jax.ShapeDtypeStruct((2 * MP,), jnp.float32),
    ],
    scratch_types=[
        pltpu.VMEM((BASE + 1, 2, C), jnp.int32),
        pltpu.VMEM((NBUF, C, D), jnp.float32),
        pltpu.VMEM((C,), jnp.float32),
        pltpu.VMEM((MP // 16,), jnp.float32),
        pltpu.VMEM_SHARED((MP, D), jnp.float32),
        pltpu.VMEM_SHARED((MP,), jnp.float32),
    ] + [pltpu.SemaphoreType.DMA] * (3 * NBUF),
)
def _sc_phase1(vfeat, idx2, z2d, z1d, ones, sums_out, cnt_out,
               idx_a, rows_v, ones_v, cnt_b, acc_sp, cnt_sp, *sems):
    cid = lax.axis_index("c")
    sid = lax.axis_index("s")
    wid = sid * 2 + cid
    rpt = MP // 16
    r0 = sid * rpt
    sg = sems[:NBUF]
    ss = sems[NBUF:2 * NBUF]
    sc = sems[2 * NBUF:]

    base0 = wid * BASE
    pltpu.sync_copy(idx2.at[pl.ds(base0, BASE)], idx_a.at[pl.ds(0, BASE)])
    pltpu.sync_copy(z2d.at[pl.ds(0, rpt)], acc_sp.at[pl.ds(r0, rpt)])
    pltpu.sync_copy(z1d.at[pl.ds(r0, rpt)], cnt_b)
    pltpu.sync_copy(cnt_b, cnt_sp.at[pl.ds(r0, rpt)])
    pltpu.sync_copy(ones, ones_v)
    plsc.subcore_barrier()

    def start_gather(k, j):
        pltpu.async_copy(vfeat.at[idx_a.at[k, 0]], rows_v.at[j], sg[j])

    def wait_gather(j):
        pltpu.make_async_copy(vfeat.at[idx_a.at[0, 0]], rows_v.at[j], sg[j]).wait()

    def wait_scatters(j):
        pltpu.make_async_copy(rows_v.at[j], acc_sp.at[idx_a.at[0, 1]], ss[j]).wait()
        pltpu.make_async_copy(ones_v, cnt_sp.at[idx_a.at[0, 1]], sc[j]).wait()

    for k in range(NBUF - 1):
        start_gather(k, k)

    def body(b, carry):
        i0 = b * NBUF
        for j in range(NBUF):
            i = i0 + j
            wait_gather(j)
            pltpu.async_copy(rows_v.at[j], acc_sp.at[idx_a.at[i, 1]], ss[j], add=True)
            pltpu.async_copy(ones_v, cnt_sp.at[idx_a.at[i, 1]], sc[j], add=True)
            jn = (j + NBUF - 1) % NBUF
            if j == 0:
                @pl.when(b > 0)
                def _():
                    wait_scatters(jn)
            else:
                wait_scatters(jn)

            @pl.when(i + NBUF - 1 < BASE)
            def _():
                start_gather(i + NBUF - 1, jn)

        return carry

    lax.fori_loop(0, NBLK, body, 0)
    wait_scatters((BASE - 1) % NBUF)

    @pl.when(wid < EXTRA)
    def _():
        pltpu.sync_copy(idx2.at[NW * BASE + wid], idx_a.at[BASE])
        pltpu.async_copy(vfeat.at[idx_a.at[BASE, 0]], rows_v.at[0], sg[0]).wait()
        pltpu.sync_copy(rows_v.at[0], acc_sp.at[idx_a.at[BASE, 1]], add=True)
        pltpu.sync_copy(ones_v, cnt_sp.at[idx_a.at[BASE, 1]], add=True)

    plsc.subcore_barrier()
    pltpu.sync_copy(acc_sp.at[pl.ds(r0, rpt)], sums_out.at[cid, pl.ds(r0, rpt)])
    pltpu.sync_copy(cnt_sp.at[pl.ds(r0, rpt)], cnt_b)
    pltpu.sync_copy(cnt_b, cnt_out.at[pl.ds(cid * MP + r0, rpt)])


@functools.partial(
    pl.kernel,
    mesh=_mesh,
    out_type=jax.ShapeDtypeStruct((2, NP, D), jnp.float32),
    scratch_types=[
        pltpu.VMEM((BASE + 1, 1, C), jnp.int32),
        pltpu.VMEM((2, 1, C), jnp.int32),
        pltpu.VMEM((2, C, D), jnp.float32),
        pltpu.VMEM_SHARED((NP, D), jnp.float32),
        pltpu.SemaphoreType.DMA, pltpu.SemaphoreType.DMA,
        pltpu.SemaphoreType.DMA, pltpu.SemaphoreType.DMA,
        pltpu.SemaphoreType.DMA, pltpu.SemaphoreType.DMA,
    ],
)
def _sc_phase2(etmp, idx_h, idx_n, z2d, vagg_out, hidx_a, sidx, rows_v, acc_sp,
               si0, si1, sg0, sg1, ss0, ss1):
    cid = lax.axis_index("c")
    sid = lax.axis_index("s")
    wid = sid * 2 + cid
    rpt = NP // 16
    r0 = sid * rpt
    si = (si0, si1)
    sg = (sg0, sg1)
    ss = (ss0, ss1)

    base0 = wid * BASE
    pltpu.sync_copy(idx_h.at[pl.ds(base0, BASE)], hidx_a.at[pl.ds(0, BASE)])
    pltpu.sync_copy(z2d.at[pl.ds(0, rpt)], acc_sp.at[pl.ds(r0, rpt)])
    plsc.subcore_barrier()

    def start_gather(k, j):
        pltpu.async_copy(etmp.at[hidx_a.at[k, 0]], rows_v.at[j], sg[j])

    def wait_gather(j):
        pltpu.make_async_copy(etmp.at[hidx_a.at[0, 0]], rows_v.at[j], sg[j]).wait()

    def wait_scatter(j):
        pltpu.make_async_copy(rows_v.at[j], acc_sp.at[sidx.at[j, 0]], ss[j]).wait()

    def wait_sidx(j):
        pltpu.make_async_copy(idx_n.at[0], sidx.at[j], si[j]).wait()

    pltpu.sync_copy(idx_n.at[base0], sidx.at[0])
    start_gather(0, 0)

    def body(b, carry):
        for j in range(2):
            i = 2 * b + j
            if j == 1:
                wait_sidx(1)
            else:
                @pl.when(b > 0)
                def _():
                    wait_sidx(0)
            wait_gather(j)
            pltpu.async_copy(rows_v.at[j], acc_sp.at[sidx.at[j, 0]], ss[j], add=True)
            if j == 0:
                @pl.when(b > 0)
                def _():
                    wait_scatter(1)

                pltpu.async_copy(idx_n.at[base0 + i + 1], sidx.at[1], si[1])
                start_gather(i + 1, 1)
            else:
                wait_scatter(0)

                @pl.when(b < NPAIR - 1)
                def _():
                    pltpu.async_copy(idx_n.at[base0 + i + 1], sidx.at[0], si[0])
                    start_gather(i + 1, 0)
        return carry

    lax.fori_loop(0, NPAIR, body, 0)
    wait_scatter((BASE - 1) % 2)

    @pl.when(wid < EXTRA)
    def _():
        c = NW * BASE + wid
        pltpu.sync_copy(idx_h.at[c], hidx_a.at[BASE])
        pltpu.sync_copy(idx_n.at[c], sidx.at[0])
        pltpu.async_copy(etmp.at[hidx_a.at[BASE, 0]], rows_v.at[0], sg[0]).wait()
        pltpu.sync_copy(rows_v.at[0], acc_sp.at[sidx.at[0, 0]], add=True)

    plsc.subcore_barrier()
    pltpu.sync_copy(acc_sp.at[pl.ds(r0, rpt)], vagg_out.at[cid, pl.ds(r0, rpt)])


def _tc1_body(sums_ref, cnt_ref, dege_ref, ef_ref, etmp_ref):
    s = sums_ref[0] + sums_ref[1]
    c = cnt_ref[0] + cnt_ref[1]
    ef = s / jnp.maximum(c, 1.0)
    ef_ref[...] = ef
    etmp_ref[...] = ef * dege_ref[...]


_tc1 = pl.pallas_call(
    _tc1_body,
    out_shape=[
        jax.ShapeDtypeStruct((MP, D), jnp.float32),
        jax.ShapeDtypeStruct((MP, D), jnp.float32),
    ],
)

RB = 1000


def _tc2_body(vagg_ref, degv_ref, vf0_ref, w_ref, ab_ref, out_ref):
    a = ab_ref[0:1, 0:1]
    b = ab_ref[0:1, 1:2]
    va = vagg_ref[0] + vagg_ref[1]
    vi = (1.0 - a) * degv_ref[...] * va + a * vf0_ref[...]
    vw = lax.dot_general(vi, w_ref[...], (((1,), (1,)), ((), ())),
                         preferred_element_type=jnp.float32,
                         precision=lax.Precision.HIGHEST)
    out_ref[...] = (1.0 - b) * vi + b * vw


_tc2 = pl.pallas_call(
    _tc2_body,
    grid=(N // RB,),
    in_specs=[
        pl.BlockSpec((2, RB, D), lambda i: (0, i, 0)),
        pl.BlockSpec((RB, 1), lambda i: (i, 0)),
        pl.BlockSpec((RB, D), lambda i: (i, 0)),
        pl.BlockSpec((D, D), lambda i: (0, 0)),
        pl.BlockSpec((1, 2), lambda i: (0, 0)),
    ],
    out_specs=pl.BlockSpec((RB, D), lambda i: (i, 0)),
    out_shape=jax.ShapeDtypeStruct((N, D), jnp.float32),
)


def kernel(vfeat, efeat, degE, degV, vfeat0, W, node_idx, hedge_idx, alpha, beta):
    del efeat
    idx2 = jnp.stack(
        [node_idx.reshape(NCHUNK, C), hedge_idx.reshape(NCHUNK, C)], axis=1)
    z2d = jnp.zeros((NP // 16, D), jnp.float32)
    z1d = jnp.zeros((MP,), jnp.float32)
    ones = jnp.ones((C,), jnp.float32)

    sums, cnt = _sc_phase1(vfeat, idx2, z2d, z1d, ones)

    dege_col = jnp.concatenate(
        [degE, jnp.zeros((MP - M,), jnp.float32)]).reshape(MP, 1)
    ef_pad, etmp_pad = _tc1(sums, cnt.reshape(2, MP, 1), dege_col)

    vagg = _sc_phase2(etmp_pad, hedge_idx.reshape(NCHUNK, 1, C),
                      node_idx.reshape(NCHUNK, 1, C), z2d)

    ab = jnp.stack([alpha, beta]).astype(jnp.float32).reshape(1, 2)
    v = _tc2(vagg, degV.reshape(N, 1), vfeat0, W, ab)
    return (v, ef_pad[:M])


# speedup vs baseline: 29.2413x; 1.0052x over previous
import functools

import jax
import jax.numpy as jnp
from jax import lax
from jax.experimental import pallas as pl
from jax.experimental.pallas import tpu as pltpu
from jax.experimental.pallas import tpu_sc as plsc

N = 10000
M = 5000
E = 320000
D = 128
MP = 5120
NP = 10240
C = 128
NCHUNK = E // C
NW = 32
BASE = NCHUNK // NW
EXTRA = NCHUNK - BASE * NW
NBUF = 4
NBLK = BASE // NBUF
TAIL = BASE - NBLK * NBUF
NPAIR = BASE // 2

_mesh = plsc.VectorSubcoreMesh(core_axis_name="c", subcore_axis_name="s")


@functools.partial(
    pl.kernel,
    mesh=_mesh,
    out_type=[
        jax.ShapeDtypeStruct((2, MP, D), jnp.float32),
        jax.ShapeDtypeStruct((2 * MP,), jnp.float32),
    ],
    scratch_types=[
        pltpu.VMEM((BASE + 1, 2, C), jnp.int32),
        pltpu.VMEM((NBUF, C, D), jnp.float32),
        pltpu.VMEM((C,), jnp.float32),
        pltpu.VMEM((MP // 16,), jnp.float32),
        pltpu.VMEM_SHARED((MP, D), jnp.float32),
        pltpu.VMEM_SHARED((MP,), jnp.float32),
    ] + [pltpu.SemaphoreType.DMA] * (3 * NBUF),
)
def _sc_phase1(vfeat, idx2, z2d, z1d, ones, sums_out, cnt_out,
               idx_a, rows_v, ones_v, cnt_b, acc_sp, cnt_sp, *sems):
    cid = lax.axis_index("c")
    sid = lax.axis_index("s")
    wid = sid * 2 + cid
    rpt = MP // 16
    r0 = sid * rpt
    sg = sems[:NBUF]
    ss = sems[NBUF:2 * NBUF]
    sc = sems[2 * NBUF:]

    base0 = wid * BASE
    pltpu.sync_copy(idx2.at[pl.ds(base0, BASE)], idx_a.at[pl.ds(0, BASE)])
    pltpu.sync_copy(z2d.at[pl.ds(0, rpt)], acc_sp.at[pl.ds(r0, rpt)])
    pltpu.sync_copy(z1d.at[pl.ds(r0, rpt)], cnt_b)
    pltpu.sync_copy(cnt_b, cnt_sp.at[pl.ds(r0, rpt)])
    pltpu.sync_copy(ones, ones_v)
    plsc.subcore_barrier()

    def start_gather(k, j):
        pltpu.async_copy(vfeat.at[idx_a.at[k, 0]], rows_v.at[j], sg[j])

    def wait_gather(j):
        pltpu.make_async_copy(vfeat.at[idx_a.at[0, 0]], rows_v.at[j], sg[j]).wait()

    def wait_scatters(j):
        pltpu.make_async_copy(rows_v.at[j], acc_sp.at[idx_a.at[0, 1]], ss[j]).wait()
        pltpu.make_async_copy(ones_v, cnt_sp.at[idx_a.at[0, 1]], sc[j]).wait()

    for k in range(NBUF - 1):
        start_gather(k, k)

    def body(b, carry):
        i0 = b * NBUF
        for j in range(NBUF):
            i = i0 + j
            wait_gather(j)
            pltpu.async_copy(rows_v.at[j], acc_sp.at[idx_a.at[i, 1]], ss[j], add=True)
            pltpu.async_copy(ones_v, cnt_sp.at[idx_a.at[i, 1]], sc[j], add=True)
            jn = (j + NBUF - 1) % NBUF
            if j == 0:
                @pl.when(b > 0)
                def _():
                    wait_scatters(jn)
            else:
                wait_scatters(jn)

            @pl.when(i + NBUF - 1 < BASE)
            def _():
                start_gather(i + NBUF - 1, jn)

        return carry

    lax.fori_loop(0, NBLK, body, 0)
    for t in range(TAIL):
        i = NBLK * NBUF + t
        wait_gather(t)
        pltpu.async_copy(rows_v.at[t], acc_sp.at[idx_a.at[i, 1]], ss[t], add=True)
        pltpu.async_copy(ones_v, cnt_sp.at[idx_a.at[i, 1]], sc[t], add=True)
        wait_scatters((t + NBUF - 1) % NBUF)
    wait_scatters((BASE - 1) % NBUF)

    @pl.when(wid < EXTRA)
    def _():
        pltpu.sync_copy(idx2.at[NW * BASE + wid], idx_a.at[BASE])
        pltpu.async_copy(vfeat.at[idx_a.at[BASE, 0]], rows_v.at[0], sg[0]).wait()
        pltpu.sync_copy(rows_v.at[0], acc_sp.at[idx_a.at[BASE, 1]], add=True)
        pltpu.sync_copy(ones_v, cnt_sp.at[idx_a.at[BASE, 1]], add=True)

    plsc.subcore_barrier()
    pltpu.sync_copy(acc_sp.at[pl.ds(r0, rpt)], sums_out.at[cid, pl.ds(r0, rpt)])
    pltpu.sync_copy(cnt_sp.at[pl.ds(r0, rpt)], cnt_b)
    pltpu.sync_copy(cnt_b, cnt_out.at[pl.ds(cid * MP + r0, rpt)])


@functools.partial(
    pl.kernel,
    mesh=_mesh,
    out_type=jax.ShapeDtypeStruct((2, NP, D), jnp.float32),
    scratch_types=[
        pltpu.VMEM((BASE + 1, 1, C), jnp.int32),
        pltpu.VMEM((2, 1, C), jnp.int32),
        pltpu.VMEM((2, C, D), jnp.float32),
        pltpu.VMEM_SHARED((NP, D), jnp.float32),
        pltpu.SemaphoreType.DMA, pltpu.SemaphoreType.DMA,
        pltpu.SemaphoreType.DMA, pltpu.SemaphoreType.DMA,
        pltpu.SemaphoreType.DMA, pltpu.SemaphoreType.DMA,
    ],
)
def _sc_phase2(etmp, idx_h, idx_n, z2d, vagg_out, hidx_a, sidx, rows_v, acc_sp,
               si0, si1, sg0, sg1, ss0, ss1):
    cid = lax.axis_index("c")
    sid = lax.axis_index("s")
    wid = sid * 2 + cid
    rpt = NP // 16
    r0 = sid * rpt
    si = (si0, si1)
    sg = (sg0, sg1)
    ss = (ss0, ss1)

    base0 = wid * BASE
    pltpu.sync_copy(idx_h.at[pl.ds(base0, BASE)], hidx_a.at[pl.ds(0, BASE)])
    pltpu.sync_copy(z2d.at[pl.ds(0, rpt)], acc_sp.at[pl.ds(r0, rpt)])
    plsc.subcore_barrier()

    def start_gather(k, j):
        pltpu.async_copy(etmp.at[hidx_a.at[k, 0]], rows_v.at[j], sg[j])

    def wait_gather(j):
        pltpu.make_async_copy(etmp.at[hidx_a.at[0, 0]], rows_v.at[j], sg[j]).wait()

    def wait_scatter(j):
        pltpu.make_async_copy(rows_v.at[j], acc_sp.at[sidx.at[j, 0]], ss[j]).wait()

    def wait_sidx(j):
        pltpu.make_async_copy(idx_n.at[0], sidx.at[j], si[j]).wait()

    pltpu.sync_copy(idx_n.at[base0], sidx.at[0])
    start_gather(0, 0)

    def body(b, carry):
        for j in range(2):
            i = 2 * b + j
            if j == 1:
                wait_sidx(1)
            else:
                @pl.when(b > 0)
                def _():
                    wait_sidx(0)
            wait_gather(j)
            pltpu.async_copy(rows_v.at[j], acc_sp.at[sidx.at[j, 0]], ss[j], add=True)
            if j == 0:
                @pl.when(b > 0)
                def _():
                    wait_scatter(1)

                pltpu.async_copy(idx_n.at[base0 + i + 1], sidx.at[1], si[1])
                start_gather(i + 1, 1)
            else:
                wait_scatter(0)

                @pl.when(b < NPAIR - 1)
                def _():
                    pltpu.async_copy(idx_n.at[base0 + i + 1], sidx.at[0], si[0])
                    start_gather(i + 1, 0)
        return carry

    lax.fori_loop(0, NPAIR, body, 0)
    wait_scatter((BASE - 1) % 2)

    @pl.when(wid < EXTRA)
    def _():
        c = NW * BASE + wid
        pltpu.sync_copy(idx_h.at[c], hidx_a.at[BASE])
        pltpu.sync_copy(idx_n.at[c], sidx.at[0])
        pltpu.async_copy(etmp.at[hidx_a.at[BASE, 0]], rows_v.at[0], sg[0]).wait()
        pltpu.sync_copy(rows_v.at[0], acc_sp.at[sidx.at[0, 0]], add=True)

    plsc.subcore_barrier()
    pltpu.sync_copy(acc_sp.at[pl.ds(r0, rpt)], vagg_out.at[cid, pl.ds(r0, rpt)])


def _tc1_body(sums_ref, cnt_ref, dege_ref, ef_ref, etmp_ref):
    s = sums_ref[0] + sums_ref[1]
    c = cnt_ref[0] + cnt_ref[1]
    ef = s / jnp.maximum(c, 1.0)
    ef_ref[...] = ef
    etmp_ref[...] = ef * dege_ref[...]


_tc1 = pl.pallas_call(
    _tc1_body,
    out_shape=[
        jax.ShapeDtypeStruct((MP, D), jnp.float32),
        jax.ShapeDtypeStruct((MP, D), jnp.float32),
    ],
)

RB = 1000


def _tc2_body(vagg_ref, degv_ref, vf0_ref, w_ref, ab_ref, out_ref):
    a = ab_ref[0:1, 0:1]
    b = ab_ref[0:1, 1:2]
    va = vagg_ref[0] + vagg_ref[1]
    vi = (1.0 - a) * degv_ref[...] * va + a * vf0_ref[...]
    vw = lax.dot_general(vi, w_ref[...], (((1,), (1,)), ((), ())),
                         preferred_element_type=jnp.float32,
                         precision=lax.Precision.HIGHEST)
    out_ref[...] = (1.0 - b) * vi + b * vw


_tc2 = pl.pallas_call(
    _tc2_body,
    grid=(N // RB,),
    in_specs=[
        pl.BlockSpec((2, RB, D), lambda i: (0, i, 0)),
        pl.BlockSpec((RB, 1), lambda i: (i, 0)),
        pl.BlockSpec((RB, D), lambda i: (i, 0)),
        pl.BlockSpec((D, D), lambda i: (0, 0)),
        pl.BlockSpec((1, 2), lambda i: (0, 0)),
    ],
    out_specs=pl.BlockSpec((RB, D), lambda i: (i, 0)),
    out_shape=jax.ShapeDtypeStruct((N, D), jnp.float32),
)


def kernel(vfeat, efeat, degE, degV, vfeat0, W, node_idx, hedge_idx, alpha, beta):
    del efeat
    idx2 = jnp.stack(
        [node_idx.reshape(NCHUNK, C), hedge_idx.reshape(NCHUNK, C)], axis=1)
    z2d = jnp.zeros((NP // 16, D), jnp.float32)
    z1d = jnp.zeros((MP,), jnp.float32)
    ones = jnp.ones((C,), jnp.float32)

    sums, cnt = _sc_phase1(vfeat, idx2, z2d, z1d, ones)

    dege_col = jnp.concatenate(
        [degE, jnp.zeros((MP - M,), jnp.float32)]).reshape(MP, 1)
    ef_pad, etmp_pad = _tc1(sums, cnt.reshape(2, MP, 1), dege_col)

    vagg = _sc_phase2(etmp_pad, hedge_idx.reshape(NCHUNK, 1, C),
                      node_idx.reshape(NCHUNK, 1, C), z2d)

    ab = jnp.stack([alpha, beta]).astype(jnp.float32).reshape(1, 2)
    v = _tc2(vagg, degV.reshape(N, 1), vfeat0, W, ab)
    return (v, ef_pad[:M])
